# baseline (speedup 1.0000x reference)
import functools

import jax
import jax.numpy as jnp
from jax import lax
from jax.experimental import pallas as pl
from jax.experimental.pallas import tpu as pltpu
from jax.experimental.pallas import tpu_sc as plsc

_B, _T, _HIDDEN = 4, 4096, 2048
_SRPE = 256
_ABS_DIM = _HIDDEN - _SRPE
_HALF = _SRPE // 2
_M = _B * _T

_CH = 128


@functools.cache
def _sc_gather():
    info = plsc.get_sparse_core_info()
    nw = info.num_cores * info.num_subcores
    rows_per_w = _M // nw
    n_ch = rows_per_w // _CH
    mesh = plsc.VectorSubcoreMesh(core_axis_name="c", subcore_axis_name="s")

    @functools.partial(
        pl.kernel,
        out_type=(
            jax.ShapeDtypeStruct((_M, _HALF), jnp.float32),
            jax.ShapeDtypeStruct((_M, _HALF), jnp.float32),
        ),
        mesh=mesh,
        scratch_types=[
            pltpu.VMEM((2, _CH), jnp.int32),
            pltpu.VMEM((2, _CH, _HALF), jnp.float32),
            [pltpu.SemaphoreType.DMA] * 2,
            [pltpu.SemaphoreType.DMA] * 2,
            [pltpu.SemaphoreType.DMA] * 2,
        ],
    )
    def gather_k(scope_hbm, depth_hbm, sidx_hbm, didx_hbm, se_out, de_out,
                 idx_v, rows_v, sem_i, sem_g, sem_o):
        wid = lax.axis_index("s") * info.num_cores + lax.axis_index("c")
        base = wid * rows_per_w
        chunks = []
        for idx_hbm, table, out in (
            (sidx_hbm, scope_hbm, se_out),
            (didx_hbm, depth_hbm, de_out),
        ):
            for c in range(n_ch):
                off = base + c * _CH
                chunks.append((idx_hbm.at[pl.ds(off, _CH)], table,
                               out.at[pl.ds(off, _CH)]))
        n = len(chunks)
        d_idx = [None, None]
        d_gat = [None, None]
        d_out = [None, None]
        for b in range(2):
            d_idx[b] = pltpu.async_copy(chunks[b][0], idx_v.at[b], sem_i[b])
        for c in range(n + 1):
            b = c % 2
            if c < n:
                d_idx[b].wait()
                if d_out[b] is not None:
                    d_out[b].wait()
                d_gat[b] = pltpu.async_copy(
                    chunks[c][1].at[idx_v.at[b]], rows_v.at[b], sem_g[b])
            if c >= 1:
                p = 1 - b
                d_gat[p].wait()
                if c + 1 < n:
                    d_idx[p] = pltpu.async_copy(chunks[c + 1][0],
                                                idx_v.at[p], sem_i[p])
                d_out[p] = pltpu.async_copy(rows_v.at[p], chunks[c - 1][2],
                                            sem_o[p])
        for b in range(2):
            d_out[b].wait()

    return gather_k


def _mm_body(x_ref, abs_ref, se_ref, de_ref, w_ref, o_ref):
    emb = jnp.concatenate(
        [abs_ref[...], se_ref[...], de_ref[...]], axis=1)
    y = (x_ref[...] + emb).astype(jnp.bfloat16)
    o_ref[...] = lax.dot_general(
        y, w_ref[...], (((1,), (0,)), ((), ())),
        preferred_element_type=jnp.float32)


@functools.cache
def _mm_call(bm):
    t_blocks = _T // bm
    row = lambda it, ib: (ib * t_blocks + it, 0)
    return pl.pallas_call(
        _mm_body,
        grid=(t_blocks, _B),
        in_specs=[
            pl.BlockSpec((bm, _HIDDEN), row),
            pl.BlockSpec((bm, _ABS_DIM), lambda it, ib: (it, 0)),
            pl.BlockSpec((bm, _HALF), row),
            pl.BlockSpec((bm, _HALF), row),
            pl.BlockSpec((_HIDDEN, _HIDDEN), lambda it, ib: (0, 0)),
        ],
        out_specs=pl.BlockSpec((bm, _HIDDEN), row),
        out_shape=jax.ShapeDtypeStruct((_M, _HIDDEN), jnp.float32),
        compiler_params=pltpu.CompilerParams(
            dimension_semantics=("arbitrary", "arbitrary")),
    )


@jax.jit
def kernel(x, scope_positions, scope_depths, abs_table, scope_table,
           depth_table, W):
    sidx = scope_positions.reshape(_M).astype(jnp.int32)
    didx = scope_depths.reshape(_M).astype(jnp.int32)
    se, de = _sc_gather()(scope_table, depth_table, sidx, didx)
    out = _mm_call(512)(
        x.reshape(_M, _HIDDEN), abs_table[:_T], se, de,
        W.astype(jnp.bfloat16).T)
    return out.reshape(_B, _T, _HIDDEN)


# speedup vs baseline: 1.0305x; 1.0305x over previous
import functools

import jax
import jax.numpy as jnp
from jax import lax
from jax.experimental import pallas as pl
from jax.experimental.pallas import tpu as pltpu
from jax.experimental.pallas import tpu_sc as plsc

_B, _T, _HIDDEN = 4, 4096, 2048
_SRPE = 256
_ABS_DIM = _HIDDEN - _SRPE
_HALF = _SRPE // 2
_M = _B * _T

_CH = 128


@functools.cache
def _sc_gather():
    info = plsc.get_sparse_core_info()
    nw = info.num_cores * info.num_subcores
    rows_per_w = _M // nw
    n_ch = rows_per_w // _CH
    mesh = plsc.VectorSubcoreMesh(core_axis_name="c", subcore_axis_name="s")

    @functools.partial(
        pl.kernel,
        out_type=(
            jax.ShapeDtypeStruct((_M, _HALF), jnp.float32),
            jax.ShapeDtypeStruct((_M, _HALF), jnp.float32),
        ),
        mesh=mesh,
        scratch_types=[
            pltpu.VMEM((2, _CH), jnp.int32),
            pltpu.VMEM((2, _CH, _HALF), jnp.float32),
            [pltpu.SemaphoreType.DMA] * 2,
            [pltpu.SemaphoreType.DMA] * 2,
            [pltpu.SemaphoreType.DMA] * 2,
        ],
    )
    def gather_k(scope_hbm, depth_hbm, sidx_hbm, didx_hbm, se_out, de_out,
                 idx_v, rows_v, sem_i, sem_g, sem_o):
        wid = lax.axis_index("s") * info.num_cores + lax.axis_index("c")
        base = wid * rows_per_w
        chunks = []
        for idx_hbm, table, out in (
            (sidx_hbm, scope_hbm, se_out),
            (didx_hbm, depth_hbm, de_out),
        ):
            for c in range(n_ch):
                off = base + c * _CH
                chunks.append((idx_hbm.at[pl.ds(off, _CH)], table,
                               out.at[pl.ds(off, _CH)]))
        n = len(chunks)
        d_idx = [None, None]
        d_gat = [None, None]
        d_out = [None, None]
        for b in range(2):
            d_idx[b] = pltpu.async_copy(chunks[b][0], idx_v.at[b], sem_i[b])
        for c in range(n + 1):
            b = c % 2
            if c < n:
                d_idx[b].wait()
                if d_out[b] is not None:
                    d_out[b].wait()
                d_gat[b] = pltpu.async_copy(
                    chunks[c][1].at[idx_v.at[b]], rows_v.at[b], sem_g[b])
            if c >= 1:
                p = 1 - b
                d_gat[p].wait()
                if c + 1 < n:
                    d_idx[p] = pltpu.async_copy(chunks[c + 1][0],
                                                idx_v.at[p], sem_i[p])
                d_out[p] = pltpu.async_copy(rows_v.at[p], chunks[c - 1][2],
                                            sem_o[p])
        for b in range(2):
            d_out[b].wait()

    return gather_k


def _mm_body(x_ref, abs_ref, se_ref, de_ref, w_ref, o_ref):
    emb = jnp.concatenate(
        [abs_ref[...].astype(jnp.float32), se_ref[...], de_ref[...]], axis=1)
    y = (x_ref[...] + emb).astype(jnp.bfloat16)
    o_ref[...] = lax.dot_general(
        y, w_ref[...], (((1,), (0,)), ((), ())),
        preferred_element_type=jnp.float32)


@functools.cache
def _mm_call(bm):
    t_blocks = _T // bm
    row = lambda it, ib: (ib * t_blocks + it, 0)
    return pl.pallas_call(
        _mm_body,
        grid=(t_blocks, _B),
        in_specs=[
            pl.BlockSpec((bm, _HIDDEN), row),
            pl.BlockSpec((bm, _ABS_DIM), lambda it, ib: (it, 0)),
            pl.BlockSpec((bm, _HALF), row),
            pl.BlockSpec((bm, _HALF), row),
            pl.BlockSpec((_HIDDEN, _HIDDEN), lambda it, ib: (0, 0)),
        ],
        out_specs=pl.BlockSpec((bm, _HIDDEN), row),
        out_shape=jax.ShapeDtypeStruct((_M, _HIDDEN), jnp.float32),
        compiler_params=pltpu.CompilerParams(
            dimension_semantics=("arbitrary", "arbitrary")),
    )


@jax.jit
def kernel(x, scope_positions, scope_depths, abs_table, scope_table,
           depth_table, W):
    sidx = scope_positions.reshape(_M).astype(jnp.int32)
    didx = scope_depths.reshape(_M).astype(jnp.int32)
    se, de = _sc_gather()(scope_table, depth_table, sidx, didx)
    out = _mm_call(1024)(
        x.reshape(_M, _HIDDEN), abs_table[:_T].astype(jnp.bfloat16), se, de,
        W.astype(jnp.bfloat16).T)
    return out.reshape(_B, _T, _HIDDEN)
